# baseline (speedup 1.0000x reference)
import functools

import jax
import jax.numpy as jnp
from jax import lax
from jax.experimental import pallas as pl
from jax.experimental.pallas import tpu as pltpu
from jax.experimental.pallas import tpu_sc as plsc

BATCH = 16384
EMBED = 64
LANES = 16

_info = plsc.get_sparse_core_info()
_NC = _info.num_cores
_NS = _info.num_subcores
_NW = _NC * _NS
_BPW = BATCH // _NW
_CH = 16
_NCH = _BPW // _CH
_BW = 16

_mesh = plsc.VectorSubcoreMesh(core_axis_name="c", subcore_axis_name="s")


@functools.partial(
    pl.kernel,
    mesh=_mesh,
    out_type=jax.ShapeDtypeStruct((EMBED, BATCH), jnp.float32),
    compiler_params=pltpu.CompilerParams(
        use_tc_tiling_on_sc=False, needs_layout_passes=False),
    scratch_types=[
        pltpu.VMEM((_BPW,), jnp.int32),
        pltpu.VMEM((_BPW,), jnp.int32),
        pltpu.VMEM((2, _CH, EMBED, _BW), jnp.float32),
        pltpu.VMEM((2, _CH, EMBED, _BW), jnp.float32),
        pltpu.VMEM((EMBED, _BPW), jnp.float32),
        pltpu.SemaphoreType.DMA,
    ],
)
def _gmf_sc(uidx_hbm, iidx_hbm, utab_hbm, itab_hbm, out_hbm,
            uidx_v, iidx_v, ublk, iblk, outb, sem):
    wid = lax.axis_index("s") * _NC + lax.axis_index("c")
    base = wid * _BPW

    pltpu.sync_copy(uidx_hbm.at[wid], uidx_v)
    pltpu.sync_copy(iidx_hbm.at[wid], iidx_v)

    def fire(ch, par):
        uv = uidx_v[pl.ds(ch * _CH, _CH)]
        iv = iidx_v[pl.ds(ch * _CH, _CH)]
        for h in range(_CH):
            cu = (uv[h] // _BW) * _BW
            pltpu.async_copy(utab_hbm.at[:, pl.ds(cu, _BW)], ublk.at[par, h], sem)
            ci = (iv[h] // _BW) * _BW
            pltpu.async_copy(itab_hbm.at[:, pl.ds(ci, _BW)], iblk.at[par, h], sem)

    def drain(par):
        for h in range(_CH):
            pltpu.make_async_copy(
                utab_hbm.at[:, pl.ds(0, _BW)], ublk.at[par, h], sem).wait()
            pltpu.make_async_copy(
                itab_hbm.at[:, pl.ds(0, _BW)], iblk.at[par, h], sem).wait()

    def extract(ch, par):
        hvec = lax.iota(jnp.int32, LANES)
        ju = uidx_v[pl.ds(ch * _CH, LANES)] % _BW
        ji = iidx_v[pl.ds(ch * _CH, LANES)] % _BW
        ub = ublk.at[par]
        ib = iblk.at[par]

        def e_body(e, _):
            evec = jnp.full((LANES,), e, jnp.int32)
            uval = plsc.load_gather(ub, [hvec, evec, ju])
            ival = plsc.load_gather(ib, [hvec, evec, ji])
            outb[e, pl.ds(ch * _CH, LANES)] = uval * ival
            return 0

        lax.fori_loop(0, EMBED, e_body, 0)

    fire(0, 0)
    fire(1, 1)

    def g_body(g2, _):
        ch = 2 * g2
        for par in range(2):
            c = ch + par
            drain(par)
            extract(c, par)

            @pl.when(c + 2 < _NCH)
            def _():
                fire(c + 2, par)
        return 0

    lax.fori_loop(0, _NCH // 2, g_body, 0)

    pltpu.sync_copy(outb, out_hbm.at[:, pl.ds(base, _BPW)])


def kernel(user_indices, item_indices, user_table, item_table):
    uidx = user_indices.astype(jnp.int32).reshape(_NW, _BPW)
    iidx = item_indices.astype(jnp.int32).reshape(_NW, _BPW)
    out_t = _gmf_sc(uidx, iidx, user_table.T, item_table.T)
    return out_t.T


# speedup vs baseline: 7.6450x; 7.6450x over previous
import functools

import jax
import jax.numpy as jnp
from jax import lax
from jax.experimental import pallas as pl
from jax.experimental.pallas import tpu as pltpu
from jax.experimental.pallas import tpu_sc as plsc

BATCH = 16384
EMBED = 64
LANES = 16

_info = plsc.get_sparse_core_info()
_NC = _info.num_cores
_NS = _info.num_subcores
_NW = _NC * _NS
_B_PER_W = BATCH // _NW
_CHUNK = 128
_NCHUNK = _B_PER_W // _CHUNK

_mesh = plsc.VectorSubcoreMesh(core_axis_name="c", subcore_axis_name="s")


@functools.partial(
    pl.kernel,
    mesh=_mesh,
    out_type=jax.ShapeDtypeStruct((BATCH, EMBED), jnp.float32),
    compiler_params=pltpu.CompilerParams(use_tc_tiling_on_sc=False),
    scratch_types=[
        pltpu.VMEM((_NCHUNK, _CHUNK), jnp.int32),
        pltpu.VMEM((_NCHUNK, _CHUNK), jnp.int32),
        pltpu.VMEM((_B_PER_W, EMBED), jnp.float32),
        pltpu.VMEM((_B_PER_W, EMBED), jnp.float32),
        pltpu.SemaphoreType.DMA,
    ],
)
def _gmf_sc(uidx_hbm, iidx_hbm, utab_hbm, itab_hbm, out_hbm,
            uidx_v, iidx_v, urows_v, irows_v, sem):
    wid = lax.axis_index("s") * _NC + lax.axis_index("c")
    base = wid * _B_PER_W

    pltpu.sync_copy(uidx_hbm.at[wid], uidx_v)
    pltpu.sync_copy(iidx_hbm.at[wid], iidx_v)

    copies = []
    for j in range(_NCHUNK):
        dst = urows_v.at[pl.ds(j * _CHUNK, _CHUNK)]
        copies.append(pltpu.async_copy(utab_hbm.at[uidx_v.at[j]], dst, sem))
    for j in range(_NCHUNK):
        dst = irows_v.at[pl.ds(j * _CHUNK, _CHUNK)]
        copies.append(pltpu.async_copy(itab_hbm.at[iidx_v.at[j]], dst, sem))
    for c in copies:
        c.wait()

    def row_body(r, _):
        for cbase in range(0, EMBED, LANES):
            sl = pl.ds(cbase, LANES)
            urows_v[r, sl] = urows_v[r, sl] * irows_v[r, sl]
        return 0

    lax.fori_loop(0, _B_PER_W, row_body, 0)

    pltpu.sync_copy(urows_v, out_hbm.at[pl.ds(base, _B_PER_W)])


def kernel(user_indices, item_indices, user_table, item_table):
    uidx = user_indices.astype(jnp.int32).reshape(_NW, _NCHUNK, _CHUNK)
    iidx = item_indices.astype(jnp.int32).reshape(_NW, _NCHUNK, _CHUNK)
    return _gmf_sc(uidx, iidx, user_table, item_table)
